# baseline (speedup 1.0000x reference)
import functools

import jax
import jax.numpy as jnp
from jax import lax
from jax.experimental import pallas as pl
from jax.experimental.pallas import tpu as pltpu
from jax.experimental.pallas import tpu_sc as plsc

_K = 1024
_RB = 8
_NB = 16
_CPP = 3
_NC = 2
_NS = 16


def _pooled(x, lab, ea1d, eb1d):
    B, C, H, W = x.shape
    E = ea1d.shape[0] // B
    groups = _NS // (B // _NC)
    cpw = C // groups
    npass = cpw // _CPP
    nch = H // _RB
    cgrp = W // 16
    rows_max = _CPP + 1
    mesh = plsc.VectorSubcoreMesh(core_axis_name="c", subcore_axis_name="s")

    @functools.partial(
        pl.kernel,
        out_type=(
            jax.ShapeDtypeStruct((B * C * E,), jnp.float32),
            jax.ShapeDtypeStruct((B * C * E,), jnp.float32),
        ),
        mesh=mesh,
        compiler_params=pltpu.CompilerParams(needs_layout_passes=False),
        scratch_types=[
            pltpu.VMEM((rows_max * _K * _NB,), jnp.float32),
            pltpu.VMEM(((cpw + 1) * _K,), jnp.float32),
            pltpu.VMEM((_RB, W), jnp.int32),
            pltpu.VMEM((_RB, W), jnp.int32),
            pltpu.VMEM((_CPP, _RB, W), jnp.float32),
            pltpu.VMEM((_CPP, _RB, W), jnp.float32),
            pltpu.VMEM((E,), jnp.int32),
            pltpu.VMEM((E,), jnp.int32),
            pltpu.VMEM((cpw * E,), jnp.float32),
            pltpu.VMEM((cpw * E,), jnp.float32),
            pltpu.SemaphoreType.DMA,
            pltpu.SemaphoreType.DMA,
        ],
    )
    def run(x_hbm, lab_hbm, ea_hbm, eb_hbm, x0_hbm, x1_hbm,
            tab_v, means_v, lab0, lab1, xv0, xv1, ea_v, eb_v, x0_v, x1_v,
            sem0, sem1):
        cid = lax.axis_index("c")
        sid = lax.axis_index("s")
        b = cid * (B // _NC) + sid // groups
        c0 = (sid % groups) * cpw
        labs = (lab0, lab1)
        xvs = (xv0, xv1)
        sems = (sem0, sem1)

        lane = lax.iota(jnp.int32, 16)
        m15 = lane == 15
        ones = jnp.ones((16,), jnp.float32)
        zero = jnp.zeros((16,), jnp.float32)
        cntm = cpw * _K

        for p in range(npass):
            rows_p = _CPP + (1 if p == 0 else 0)
            cp0 = c0 + p * _CPP

            @pl.loop(0, rows_p * _K * _NB // 16)
            def _zero(i):
                tab_v[pl.ds(i * 16, 16)] = zero

            def copies(ch, par):
                h0 = ch * _RB
                return (
                    pltpu.make_async_copy(
                        lab_hbm.at[b, 0, pl.ds(h0, _RB), :], labs[par],
                        sems[par]),
                    pltpu.make_async_copy(
                        x_hbm.at[b, pl.ds(cp0, _CPP), pl.ds(h0, _RB), :],
                        xvs[par], sems[par]),
                )

            for cp in copies(0, 0):
                cp.start()

            def scat(labv, vals):
                lb16 = (labv * 16) + lane
                if p == 0:
                    plsc.addupdate_scatter(
                        tab_v, [lb16 + (_CPP * _K * _NB)], ones)
                for j in range(_CPP):
                    plsc.addupdate_scatter(
                        tab_v, [lb16 + (j * _K * _NB)], vals[j])

            @pl.loop(0, nch, step=2)
            def _chunk(ch2):
                for par in range(2):
                    ch = ch2 + par
                    for cp in copies(ch, par):
                        cp.wait()

                    @pl.when(ch + 1 < nch)
                    def _pref():
                        for cp in copies(ch + 1, 1 - par):
                            cp.start()

                    lab_v = labs[par]
                    xv = xvs[par]

                    @pl.loop(0, _RB)
                    def _row(r):
                        def load_grp(gi):
                            i16 = gi * 16
                            return (
                                lab_v[r, pl.ds(i16, 16)],
                                tuple(xv[j, r, pl.ds(i16, 16)]
                                      for j in range(_CPP)),
                            )

                        @plsc.parallel_loop(1, cgrp, unroll=2,
                                            carry=load_grp(0))
                        def _grp(gi, c):
                            nxt = load_grp(gi)
                            scat(c[0], c[1])
                            return nxt

                        scat(_grp[0], _grp[1])

            for t in range(rows_p):
                dst_row = cntm if (p == 0 and t == _CPP) else (cp0 - c0 + t) * _K

                @pl.loop(0, _K, unroll=8)
                def _red(k):
                    v = tab_v[pl.ds((t * _K + k) * _NB, 16)]
                    s = plsc.cumsum(v)
                    plsc.store_scatter(
                        means_v, [jnp.full((16,), dst_row, jnp.int32) + k],
                        s, mask=m15)

        @pl.loop(0, _K // 16)
        def _means(i):
            k16 = i * 16
            r = 1.0 / means_v[pl.ds(cntm + k16, 16)]
            for j in range(cpw):
                means_v[pl.ds(j * _K + k16, 16)] = (
                    means_v[pl.ds(j * _K + k16, 16)] * r)

        pltpu.sync_copy(ea_hbm.at[pl.ds(b * E, E)], ea_v)
        pltpu.sync_copy(eb_hbm.at[pl.ds(b * E, E)], eb_v)

        @pl.loop(0, E // 16)
        def _edges(e):
            e16 = e * 16
            ia = ea_v[pl.ds(e16, 16)]
            ib = eb_v[pl.ds(e16, 16)]
            for j in range(cpw):
                x0_v[pl.ds(j * E + e16, 16)] = plsc.load_gather(
                    means_v, [ia + j * _K])
                x1_v[pl.ds(j * E + e16, 16)] = plsc.load_gather(
                    means_v, [ib + j * _K])

        obase = (b * C + c0) * E
        pltpu.sync_copy(x0_v, x0_hbm.at[pl.ds(obase, cpw * E)])
        pltpu.sync_copy(x1_v, x1_hbm.at[pl.ds(obase, cpw * E)])

    return run(x, lab, ea1d, eb1d)


def kernel(x, graphs, label_maps, edges_to_pool):
    B, C, H, W = x.shape
    E = edges_to_pool.shape[1]
    ea1d = edges_to_pool[:, :, 0].reshape(-1)
    eb1d = edges_to_pool[:, :, 1].reshape(-1)
    y = edges_to_pool[:, :, 2].astype(jnp.float32)

    x0f, x1f = _pooled(x, label_maps, ea1d, eb1d)
    x0 = x0f.reshape(B, C, E).transpose(0, 2, 1)
    x1 = x1f.reshape(B, C, E).transpose(0, 2, 1)
    return x0, x1, y


# speedup vs baseline: 1.6527x; 1.6527x over previous
import functools

import jax
import jax.numpy as jnp
from jax import lax
from jax.experimental import pallas as pl
from jax.experimental.pallas import tpu as pltpu
from jax.experimental.pallas import tpu_sc as plsc

_K = 1024
_RB = 8
_NC = 2
_NS = 16
_NW = _NC * _NS


def _pooled(x, lab, ea1d, eb1d):
    B, C, H, W = x.shape
    E = ea1d.shape[0] // B
    groups = _NW // B
    cpw = C // groups
    nch = H // _RB
    mesh = plsc.VectorSubcoreMesh(core_axis_name="c", subcore_axis_name="s")

    @functools.partial(
        pl.kernel,
        out_type=(
            jax.ShapeDtypeStruct((B * C * E,), jnp.float32),
            jax.ShapeDtypeStruct((B * C * E,), jnp.float32),
        ),
        mesh=mesh,
        compiler_params=pltpu.CompilerParams(needs_layout_passes=False),
        scratch_types=[
            pltpu.VMEM(((cpw + 1) * _K,), jnp.float32),
            pltpu.VMEM((_RB, W), jnp.int32),
            pltpu.VMEM((_RB, W), jnp.int32),
            pltpu.VMEM((cpw, _RB, W), jnp.float32),
            pltpu.VMEM((cpw, _RB, W), jnp.float32),
            pltpu.VMEM((E,), jnp.int32),
            pltpu.VMEM((E,), jnp.int32),
            pltpu.VMEM((cpw * E,), jnp.float32),
            pltpu.VMEM((cpw * E,), jnp.float32),
            pltpu.SemaphoreType.DMA,
            pltpu.SemaphoreType.DMA,
        ],
    )
    def run(x_hbm, lab_hbm, ea_hbm, eb_hbm, x0_hbm, x1_hbm,
            sums_v, lab0, lab1, xv0, xv1, ea_v, eb_v, x0_v, x1_v,
            sem0, sem1):
        wid = lax.axis_index("s") * _NC + lax.axis_index("c")
        b = wid % B
        c0 = (wid // B) * cpw
        cnt_base = cpw * _K
        labs = (lab0, lab1)
        xvs = (xv0, xv1)
        sems = (sem0, sem1)

        def copies(ch, par):
            h0 = ch * _RB
            return (
                pltpu.make_async_copy(
                    lab_hbm.at[b, 0, pl.ds(h0, _RB), :], labs[par], sems[par]),
                pltpu.make_async_copy(
                    x_hbm.at[b, pl.ds(c0, cpw), pl.ds(h0, _RB), :],
                    xvs[par], sems[par]),
            )

        zero = jnp.zeros((16,), jnp.float32)

        @pl.loop(0, (cpw + 1) * _K // 16)
        def _zero(i):
            sums_v[pl.ds(i * 16, 16)] = zero

        ones = jnp.ones((16,), jnp.float32)
        cgrp = W // 16

        for cp in copies(0, 0):
            cp.start()

        @pl.loop(0, nch, step=2)
        def _chunk(ch2):
            for par in range(2):
                ch = ch2 + par
                for cp in copies(ch, par):
                    cp.wait()

                @pl.when(ch + 1 < nch)
                def _pref():
                    for cp in copies(ch + 1, 1 - par):
                        cp.start()

                lab_v = labs[par]
                xv = xvs[par]

                def scat(labv, vals):
                    plsc.addupdate_scatter(sums_v, [labv + cnt_base], ones)
                    for j in range(cpw):
                        plsc.addupdate_scatter(
                            sums_v, [labv + (j * _K)], vals[j])

                @pl.loop(0, _RB)
                def _row(r):
                    def load_grp(gi):
                        i16 = gi * 16
                        labv = lab_v[r, pl.ds(i16, 16)]
                        vals = tuple(
                            xv[j, r, pl.ds(i16, 16)] for j in range(cpw))
                        return (labv, vals)

                    @plsc.parallel_loop(1, cgrp, unroll=2,
                                        carry=load_grp(0))
                    def _grp(gi, c):
                        nxt = load_grp(gi)
                        scat(c[0], c[1])
                        return nxt

                    scat(_grp[0], _grp[1])

        @pl.loop(0, _K // 16)
        def _means(i):
            k16 = i * 16
            r = 1.0 / sums_v[pl.ds(cnt_base + k16, 16)]
            for j in range(cpw):
                sums_v[pl.ds(j * _K + k16, 16)] = (
                    sums_v[pl.ds(j * _K + k16, 16)] * r)

        pltpu.sync_copy(ea_hbm.at[pl.ds(b * E, E)], ea_v)
        pltpu.sync_copy(eb_hbm.at[pl.ds(b * E, E)], eb_v)

        @pl.loop(0, E // 16)
        def _edges(e):
            e16 = e * 16
            ia = ea_v[pl.ds(e16, 16)]
            ib = eb_v[pl.ds(e16, 16)]
            for j in range(cpw):
                x0_v[pl.ds(j * E + e16, 16)] = plsc.load_gather(
                    sums_v, [ia + j * _K])
                x1_v[pl.ds(j * E + e16, 16)] = plsc.load_gather(
                    sums_v, [ib + j * _K])

        obase = (b * C + c0) * E
        pltpu.sync_copy(x0_v, x0_hbm.at[pl.ds(obase, cpw * E)])
        pltpu.sync_copy(x1_v, x1_hbm.at[pl.ds(obase, cpw * E)])

    return run(x, lab, ea1d, eb1d)


def kernel(x, graphs, label_maps, edges_to_pool):
    B, C, H, W = x.shape
    E = edges_to_pool.shape[1]
    ea1d = edges_to_pool[:, :, 0].reshape(-1)
    eb1d = edges_to_pool[:, :, 1].reshape(-1)
    y = edges_to_pool[:, :, 2].astype(jnp.float32)

    x0f, x1f = _pooled(x, label_maps, ea1d, eb1d)
    x0 = x0f.reshape(B, C, E).transpose(0, 2, 1)
    x1 = x1f.reshape(B, C, E).transpose(0, 2, 1)
    return x0, x1, y
